# baseline (speedup 1.0000x reference)
import numpy as np
import jax
import jax.numpy as jnp
from jax.experimental import pallas as pl

B = 8
CELL = 8
NUM_CLASSES = 4
DIST_TH = 10.0
VERT_TH = 0.015
MAXV = 300
FDIM = 256
POS_FREQ = 10
SINK_ITERS = 10
GNN_LAYERS = 7
HEADS = 4
Hc, Wc = 25, 50
H, W = Hc * CELL, Wc * CELL
PE_DIM = 2 + 2 * 2 * POS_FREQ
DT_DIM = (NUM_CLASSES - 1) * CELL * CELL
NEG = -1e9


def _max_pool2d(x, r):
    h, w = x.shape
    ninf = jnp.full((r, w), -jnp.inf, x.dtype)
    xp = jnp.concatenate([ninf, x, ninf], axis=0)
    x = xp[0:h, :]
    for k in range(1, 2 * r + 1):
        x = jnp.maximum(x, xp[k:k + h, :])
    ninf = jnp.full((h, r), -jnp.inf, x.dtype)
    xp = jnp.concatenate([ninf, x, ninf], axis=1)
    x = xp[:, 0:w]
    for k in range(1, 2 * r + 1):
        x = jnp.maximum(x, xp[:, k:k + w])
    return x


def _stage1_kernel(vertex_ref, s_ref, row_ref, col_ref, mask_ref):
    vals, pixfs = [], []
    for b in range(B):
        v = vertex_ref[b]
        v = v - jnp.max(v, axis=0, keepdims=True)
        e = jnp.exp(v)
        sm = e / jnp.sum(e, axis=0, keepdims=True)
        sc = sm[:-1]
        mval = jnp.max(sc, axis=0)
        j0 = jax.lax.broadcasted_iota(
            jnp.int32, (CELL * CELL, Hc, Wc), 0).astype(jnp.float32)
        eq = sc == mval[None]
        mind = jnp.min(jnp.where(eq, j0, 64.0), axis=0)

        def _up8(x):
            x = jnp.broadcast_to(x[:, None, :], (Hc, CELL, Wc)).reshape(H, Wc)
            x = jnp.broadcast_to(x[:, :, None], (H, Wc, CELL)).reshape(H, W)
            return x

        mval_up = _up8(mval)
        mind_up = _up8(mind)
        yy = jax.lax.broadcasted_iota(jnp.int32, (H, W), 0)
        xx = jax.lax.broadcasted_iota(jnp.int32, (H, W), 1)
        jmap = ((yy % CELL) * CELL + (xx % CELL)).astype(jnp.float32)
        scores = jnp.where(mind_up == jmap, mval_up, 0.0)

        r = CELL // 2
        max_mask = scores == _max_pool2d(scores, r)
        for _ in range(2):
            supp_mask = _max_pool2d(max_mask.astype(jnp.float32), r) > 0
            supp_scores = jnp.where(supp_mask, 0.0, scores)
            new_max = supp_scores == _max_pool2d(supp_scores, r)
            max_mask = max_mask | (new_max & (~supp_mask))
        scores = jnp.where(max_mask, scores, 0.0)

        u = jnp.max(scores.reshape(Hc, CELL, W), axis=1)
        ut = u.T
        vt = jnp.max(ut.reshape(Wc, CELL, Hc), axis=1)
        val = vt.T
        ci = jax.lax.broadcasted_iota(jnp.int32, (Hc, Wc), 0)
        cj = jax.lax.broadcasted_iota(jnp.int32, (Hc, Wc), 1)
        mi = mind.astype(jnp.int32)
        prow = ci * CELL + mi // CELL
        pcol = cj * CELL + mi % CELL
        vals.append(val[None])
        pixfs.append((prow * W + pcol)[None])

    val = jnp.concatenate(vals, axis=0)
    pixf = jnp.concatenate(pixfs, axis=0)

    BIGI = jnp.int32(H * W)
    lane = jax.lax.broadcasted_iota(jnp.int32, (1, MAXV), 1)

    def body(i, carry):
        s, os_, or_, oc_, om_ = carry
        m3 = jnp.max(jnp.max(s, axis=2, keepdims=True),
                     axis=1, keepdims=True)
        m2 = jnp.max(jnp.max(s, axis=2), axis=1, keepdims=True)
        w_ = jnp.where(s == m3, pixf, BIGI)
        idx3 = jnp.min(jnp.min(w_, axis=2, keepdims=True),
                       axis=1, keepdims=True)
        idx2 = jnp.min(jnp.min(w_, axis=2), axis=1, keepdims=True)
        keep = m2 > VERT_TH
        sel = (lane == i) & keep
        os_ = jnp.where(sel, m2, os_)
        or_ = jnp.where(sel, idx2 // W, or_)
        oc_ = jnp.where(sel, idx2 % W, oc_)
        om_ = jnp.where(sel, 1.0, om_)
        s = jnp.where(pixf == idx3, -1.0, s)
        return (s, os_, or_, oc_, om_)

    z = jnp.zeros((B, MAXV), jnp.float32)
    zi = jnp.zeros((B, MAXV), jnp.int32)
    _, os_, or_, oc_, om_ = jax.lax.fori_loop(
        0, MAXV, body, (val, z, zi, zi, z))
    s_ref[...] = os_
    row_ref[...] = or_
    col_ref[...] = oc_
    mask_ref[...] = om_


def _run_stage1(vertex):
    out = pl.pallas_call(
        _stage1_kernel,
        out_shape=[
            jax.ShapeDtypeStruct((B, MAXV), jnp.float32),
            jax.ShapeDtypeStruct((B, MAXV), jnp.int32),
            jax.ShapeDtypeStruct((B, MAXV), jnp.int32),
            jax.ShapeDtypeStruct((B, MAXV), jnp.float32),
        ],
    )(vertex)
    return out


def _bn_relu(x, g, b):
    m = jnp.mean(x, axis=0, keepdims=True)
    v = jnp.mean((x - m) * (x - m), axis=0, keepdims=True)
    x = (x - m) / jnp.sqrt(v + 1e-5) * g + b
    return jnp.maximum(x, 0.0)


def _lin(x, W_, b_):
    return jnp.dot(x, W_.T, preferred_element_type=jnp.float32) + b_


def _mlp(x, layers):
    n = len(layers)
    for i, (W_, b_, g_, be_) in enumerate(layers):
        x = _lin(x, W_, b_)
        if i < n - 1:
            x = _bn_relu(x, g_, be_)
    return x


def _logsumexp(x, axis):
    m = jnp.max(x, axis=axis, keepdims=True)
    return m + jnp.log(jnp.sum(jnp.exp(x - m), axis=axis, keepdims=True))


def _make_stage2_kernel(n_weights):
    def _stage2_kernel(s_in, row_in, col_in, mask_in, dtc_in, *refs):
        wrefs = refs[:n_weights]
        (logcls_ref, matches_ref) = refs[n_weights:]
        cursor = [0]

        def nxt():
            r = wrefs[cursor[0]][...]
            cursor[0] += 1
            return r

        venc = [(nxt(), nxt(), nxt(), nxt()) for _ in range(4)]
        dtenc = [(nxt(), nxt(), nxt(), nxt()) for _ in range(4)]
        gnn = []
        for _ in range(GNN_LAYERS):
            qkv = [(nxt(), nxt()) for _ in range(3)]
            merge = (nxt(), nxt())
            mlp = [(nxt(), nxt(), nxt(), nxt()) for _ in range(2)]
            gnn.append((qkv, merge, mlp))
        fproj = (nxt(), nxt())
        chead = (nxt(), nxt())
        alpha = nxt()[0]

        top_s = s_in[...]
        rows = row_in[...].astype(jnp.float32)
        cols = col_in[...].astype(jnp.float32)
        vmask = mask_in[...]

        cell_idx = (row_in[...] // CELL) * Wc + (col_in[...] // CELL)
        ncell = Hc * Wc
        onehot = (cell_idx[:, :, None] ==
                  jax.lax.broadcasted_iota(jnp.int32, (1, 1, ncell), 2)
                  ).astype(jnp.float32)
        dtc = dtc_in[...]
        dtc = jnp.clip(jnp.maximum(dtc, 0.0), 0.0, DIST_TH)
        dt_emb = jax.lax.dot_general(
            onehot, dtc, (((2,), (1,)), ((0,), (0,))),
            preferred_element_type=jnp.float32)
        dt_emb = dt_emb * vmask[:, :, None]

        vx = (cols - W / 2.0) / (0.7 * float(W))
        vy = (rows - H / 2.0) / (0.7 * float(W))
        pe = [vx[..., None], vy[..., None]]
        for i in range(POS_FREQ):
            f = 2.0 ** i
            pe.append(jnp.sin(vx * f)[..., None])
            pe.append(jnp.sin(vy * f)[..., None])
            pe.append(jnp.cos(vx * f)[..., None])
            pe.append(jnp.cos(vy * f)[..., None])
        pe.append(top_s[..., None])
        pos_emb = jnp.concatenate(pe, axis=-1)

        BN = B * MAXV
        x = _mlp(pos_emb.reshape(BN, -1), venc) + _mlp(dt_emb.reshape(BN, -1), dtenc)

        kmask = jnp.broadcast_to(vmask[:, None, :], (B, MAXV, MAXV))

        for (qkv, merge, mlp) in gnn:
            q = _lin(x, qkv[0][0], qkv[0][1]).reshape(B, MAXV, FDIM)
            k = _lin(x, qkv[1][0], qkv[1][1]).reshape(B, MAXV, FDIM)
            v = _lin(x, qkv[2][0], qkv[2][1]).reshape(B, MAXV, FDIM)
            hd = FDIM // HEADS
            msgs = []
            for h in range(HEADS):
                qh = q[:, :, h * hd:(h + 1) * hd]
                kh = k[:, :, h * hd:(h + 1) * hd]
                vh = v[:, :, h * hd:(h + 1) * hd]
                s = jax.lax.dot_general(
                    qh, kh, (((2,), (2,)), ((0,), (0,))),
                    preferred_element_type=jnp.float32) / (hd ** 0.5)
                s = jnp.where(kmask == 0.0, NEG, s)
                s = s - jnp.max(s, axis=-1, keepdims=True)
                es = jnp.exp(s)
                p = es / jnp.sum(es, axis=-1, keepdims=True)
                msgs.append(jax.lax.dot_general(
                    p, vh, (((2,), (1,)), ((0,), (0,))),
                    preferred_element_type=jnp.float32))
            msg = jnp.concatenate(msgs, axis=-1).reshape(BN, FDIM)
            msg = _lin(msg, merge[0], merge[1])
            y = jnp.concatenate([x, msg], axis=-1)
            x = x + _mlp(y, mlp)

        ge = _lin(x, fproj[0], fproj[1])
        cls = _lin(ge, chead[0], chead[1])
        lcls = cls - _logsumexp(cls, axis=-1)
        logcls_ref[...] = lcls.reshape(B, MAXV, NUM_CLASSES - 1)

        geb = ge.reshape(B, MAXV, FDIM)
        mt = jax.lax.dot_general(
            geb, geb, (((2,), (2,)), ((0,), (0,))),
            preferred_element_type=jnp.float32) / (FDIM ** 0.5)
        ii = jax.lax.broadcasted_iota(jnp.int32, (1, MAXV, MAXV), 1)
        jj = jax.lax.broadcasted_iota(jnp.int32, (1, MAXV, MAXV), 2)
        mt = jnp.where(ii == jj, NEG, mt)
        mm = vmask[:, :, None] * vmask[:, None, :]
        mt = jnp.where(mm == 0.0, NEG, mt)

        m = n = MAXV
        acol = jnp.broadcast_to(alpha, (B, m, 1)).astype(jnp.float32)
        arow = jnp.broadcast_to(alpha, (B, 1, n + 1)).astype(jnp.float32)
        couplings = jnp.concatenate(
            [jnp.concatenate([mt, acol], axis=2), arow], axis=1)
        norm = -float(np.log(m + n))
        pi = jax.lax.broadcasted_iota(jnp.int32, (B, m + 1, 1), 1)
        pj = jax.lax.broadcasted_iota(jnp.int32, (B, 1, n + 1), 2)
        lrow = jnp.where(pi < m, norm, float(np.log(n)) + norm)
        lcol = jnp.where(pj < n, norm, float(np.log(m)) + norm)
        u = jnp.zeros((B, m + 1, 1), jnp.float32)
        v = jnp.zeros((B, 1, n + 1), jnp.float32)
        for _ in range(SINK_ITERS):
            u = lrow - _logsumexp(couplings + v, axis=2)
            v = lcol - _logsumexp(couplings + u, axis=1)
        matches_ref[...] = couplings + u + v - norm

    return _stage2_kernel


def _flatten_params(params):
    flat = []
    for ly in params['venc']:
        flat += [ly['W'], ly['b'], ly['gamma'], ly['beta']]
    for ly in params['dtenc']:
        flat += [ly['W'], ly['b'], ly['gamma'], ly['beta']]
    perm = np.array([(kk % (FDIM // HEADS)) * HEADS + kk // (FDIM // HEADS)
                     for kk in range(FDIM)])
    for layer in params['gnn']:
        for nm in ('q', 'k', 'v'):
            flat += [layer[nm]['W'][perm, :], layer[nm]['b'][perm]]
        flat += [layer['merge']['W'][:, perm], layer['merge']['b']]
        for ly in layer['mlp']:
            flat += [ly['W'], ly['b'], ly['gamma'], ly['beta']]
    flat += [params['final_proj']['W'], params['final_proj']['b']]
    flat += [params['cls_head']['W'], params['cls_head']['b']]
    flat += [params['bin_score'].reshape(1)]
    return flat


def _run_stage2(top_s, rows_i, cols_i, vmask, dtc_flat, params):
    wflat = _flatten_params(params)
    n_w = len(wflat)
    out = pl.pallas_call(
        _make_stage2_kernel(n_w),
        out_shape=[
            jax.ShapeDtypeStruct((B, MAXV, NUM_CLASSES - 1), jnp.float32),
            jax.ShapeDtypeStruct((B, MAXV + 1, MAXV + 1), jnp.float32),
        ],
    )(top_s, rows_i, cols_i, vmask, dtc_flat, *wflat)
    return out


def kernel(semantic, distance, vertex, instance, direction, params):
    top_s, rows_i, cols_i, vmask = _run_stage1(vertex)

    dtc_flat = distance.reshape(B, NUM_CLASSES - 1, Hc, CELL, Wc, CELL)
    dtc_flat = dtc_flat.transpose(0, 2, 4, 1, 3, 5).reshape(B, Hc * Wc, DT_DIM)

    lcls, matches = _run_stage2(top_s, rows_i, cols_i, vmask, dtc_flat, params)

    log_cls = lcls.transpose(0, 2, 1)
    vertices = jnp.stack([cols_i, rows_i], axis=-1)
    masks = vmask[:, :, None]
    return (log_cls, distance, vertex, instance, direction, matches,
            vertices, masks)


# speedup vs baseline: 1.0766x; 1.0766x over previous
import numpy as np
import jax
import jax.numpy as jnp
from jax.experimental import pallas as pl

B = 8
CELL = 8
NUM_CLASSES = 4
DIST_TH = 10.0
VERT_TH = 0.015
MAXV = 300
FDIM = 256
POS_FREQ = 10
SINK_ITERS = 10
GNN_LAYERS = 7
HEADS = 4
Hc, Wc = 25, 50
H, W = Hc * CELL, Wc * CELL
PE_DIM = 2 + 2 * 2 * POS_FREQ
DT_DIM = (NUM_CLASSES - 1) * CELL * CELL
NEG = -1e9


def _max_pool2d(x, r):
    h, w = x.shape
    ninf = jnp.full((r, w), -jnp.inf, x.dtype)
    xp = jnp.concatenate([ninf, x, ninf], axis=0)
    x = xp[0:h, :]
    for k in range(1, 2 * r + 1):
        x = jnp.maximum(x, xp[k:k + h, :])
    ninf = jnp.full((h, r), -jnp.inf, x.dtype)
    xp = jnp.concatenate([ninf, x, ninf], axis=1)
    x = xp[:, 0:w]
    for k in range(1, 2 * r + 1):
        x = jnp.maximum(x, xp[:, k:k + w])
    return x


def _stage1_kernel(vertex_ref, s_ref, row_ref, col_ref, mask_ref):
    vals, pixfs = [], []
    for b in range(B):
        v = vertex_ref[b]
        v = v - jnp.max(v, axis=0, keepdims=True)
        e = jnp.exp(v)
        sm = e / jnp.sum(e, axis=0, keepdims=True)
        sc = sm[:-1]
        mval = jnp.max(sc, axis=0)
        j0 = jax.lax.broadcasted_iota(
            jnp.int32, (CELL * CELL, Hc, Wc), 0).astype(jnp.float32)
        eq = sc == mval[None]
        mind = jnp.min(jnp.where(eq, j0, 64.0), axis=0)

        def _up8(x):
            x = jnp.broadcast_to(x[:, None, :], (Hc, CELL, Wc)).reshape(H, Wc)
            x = jnp.broadcast_to(x[:, :, None], (H, Wc, CELL)).reshape(H, W)
            return x

        mval_up = _up8(mval)
        mind_up = _up8(mind)
        yy = jax.lax.broadcasted_iota(jnp.int32, (H, W), 0)
        xx = jax.lax.broadcasted_iota(jnp.int32, (H, W), 1)
        jmap = ((yy % CELL) * CELL + (xx % CELL)).astype(jnp.float32)
        scores = jnp.where(mind_up == jmap, mval_up, 0.0)

        r = CELL // 2
        max_mask = scores == _max_pool2d(scores, r)
        for _ in range(2):
            supp_mask = _max_pool2d(max_mask.astype(jnp.float32), r) > 0
            supp_scores = jnp.where(supp_mask, 0.0, scores)
            new_max = supp_scores == _max_pool2d(supp_scores, r)
            max_mask = max_mask | (new_max & (~supp_mask))
        scores = jnp.where(max_mask, scores, 0.0)

        u = jnp.max(scores.reshape(Hc, CELL, W), axis=1)
        ut = u.T
        vt = jnp.max(ut.reshape(Wc, CELL, Hc), axis=1)
        val = vt.T
        ci = jax.lax.broadcasted_iota(jnp.int32, (Hc, Wc), 0)
        cj = jax.lax.broadcasted_iota(jnp.int32, (Hc, Wc), 1)
        mi = mind.astype(jnp.int32)
        prow = ci * CELL + mi // CELL
        pcol = cj * CELL + mi % CELL
        vals.append(val[None])
        pixfs.append((prow * W + pcol)[None])

    val = jnp.concatenate(vals, axis=0)
    pixf = jnp.concatenate(pixfs, axis=0)

    BIGI = jnp.int32(H * W)
    lane = jax.lax.broadcasted_iota(jnp.int32, (1, MAXV), 1)

    def body(i, carry):
        s, os_, oi_, om_ = carry
        m3 = jnp.max(jnp.max(s, axis=2, keepdims=True),
                     axis=1, keepdims=True)
        w_ = jnp.where(s == m3, pixf, BIGI)
        idx3 = jnp.min(jnp.min(w_, axis=2, keepdims=True),
                       axis=1, keepdims=True)
        m2 = jnp.max(m3, axis=2)
        idx2 = jnp.min(idx3, axis=2)
        keep = m2 > VERT_TH
        sel = (lane == i) & keep
        os_ = jnp.where(sel, m2, os_)
        oi_ = jnp.where(sel, idx2, oi_)
        om_ = jnp.where(sel, 1.0, om_)
        s = jnp.where(pixf == idx3, -1.0, s)
        return (s, os_, oi_, om_)

    z = jnp.zeros((B, MAXV), jnp.float32)
    zi = jnp.zeros((B, MAXV), jnp.int32)
    _, os_, oi_, om_ = jax.lax.fori_loop(
        0, MAXV, body, (val, z, zi, z))
    s_ref[...] = os_
    row_ref[...] = oi_ // W
    col_ref[...] = oi_ % W
    mask_ref[...] = om_


def _run_stage1(vertex):
    out = pl.pallas_call(
        _stage1_kernel,
        out_shape=[
            jax.ShapeDtypeStruct((B, MAXV), jnp.float32),
            jax.ShapeDtypeStruct((B, MAXV), jnp.int32),
            jax.ShapeDtypeStruct((B, MAXV), jnp.int32),
            jax.ShapeDtypeStruct((B, MAXV), jnp.float32),
        ],
    )(vertex)
    return out


def _bn_relu(x, g, b):
    m = jnp.mean(x, axis=0, keepdims=True)
    v = jnp.mean((x - m) * (x - m), axis=0, keepdims=True)
    x = (x - m) / jnp.sqrt(v + 1e-5) * g + b
    return jnp.maximum(x, 0.0)


def _lin(x, W_, b_):
    return jnp.dot(x, W_.T, preferred_element_type=jnp.float32) + b_


def _mlp(x, layers):
    n = len(layers)
    for i, (W_, b_, g_, be_) in enumerate(layers):
        x = _lin(x, W_, b_)
        if i < n - 1:
            x = _bn_relu(x, g_, be_)
    return x


def _logsumexp(x, axis):
    m = jnp.max(x, axis=axis, keepdims=True)
    return m + jnp.log(jnp.sum(jnp.exp(x - m), axis=axis, keepdims=True))


def _make_stage2_kernel(n_weights):
    def _stage2_kernel(s_in, row_in, col_in, mask_in, dtc_in, *refs):
        wrefs = refs[:n_weights]
        (logcls_ref, matches_ref) = refs[n_weights:]
        cursor = [0]

        def nxt():
            r = wrefs[cursor[0]][...]
            cursor[0] += 1
            return r

        venc = [(nxt(), nxt(), nxt(), nxt()) for _ in range(4)]
        dtenc = [(nxt(), nxt(), nxt(), nxt()) for _ in range(4)]
        gnn = []
        for _ in range(GNN_LAYERS):
            qkv = [(nxt(), nxt()) for _ in range(3)]
            merge = (nxt(), nxt())
            mlp = [(nxt(), nxt(), nxt(), nxt()) for _ in range(2)]
            gnn.append((qkv, merge, mlp))
        fproj = (nxt(), nxt())
        chead = (nxt(), nxt())
        alpha = nxt()[0]

        top_s = s_in[...]
        rows = row_in[...].astype(jnp.float32)
        cols = col_in[...].astype(jnp.float32)
        vmask = mask_in[...]

        cell_idx = (row_in[...] // CELL) * Wc + (col_in[...] // CELL)
        ncell = Hc * Wc
        onehot = (cell_idx[:, :, None] ==
                  jax.lax.broadcasted_iota(jnp.int32, (1, 1, ncell), 2)
                  ).astype(jnp.float32)
        dtc = dtc_in[...]
        dtc = jnp.clip(jnp.maximum(dtc, 0.0), 0.0, DIST_TH)
        dt_emb = jax.lax.dot_general(
            onehot, dtc, (((2,), (1,)), ((0,), (0,))),
            preferred_element_type=jnp.float32)
        dt_emb = dt_emb * vmask[:, :, None]

        vx = (cols - W / 2.0) / (0.7 * float(W))
        vy = (rows - H / 2.0) / (0.7 * float(W))
        pe = [vx[..., None], vy[..., None]]
        for i in range(POS_FREQ):
            f = 2.0 ** i
            pe.append(jnp.sin(vx * f)[..., None])
            pe.append(jnp.sin(vy * f)[..., None])
            pe.append(jnp.cos(vx * f)[..., None])
            pe.append(jnp.cos(vy * f)[..., None])
        pe.append(top_s[..., None])
        pos_emb = jnp.concatenate(pe, axis=-1)

        BN = B * MAXV
        x = _mlp(pos_emb.reshape(BN, -1), venc) + _mlp(dt_emb.reshape(BN, -1), dtenc)

        kmask = jnp.broadcast_to(vmask[:, None, :], (B, MAXV, MAXV))

        for (qkv, merge, mlp) in gnn:
            q = _lin(x, qkv[0][0], qkv[0][1]).reshape(B, MAXV, FDIM)
            k = _lin(x, qkv[1][0], qkv[1][1]).reshape(B, MAXV, FDIM)
            v = _lin(x, qkv[2][0], qkv[2][1]).reshape(B, MAXV, FDIM)
            hd = FDIM // HEADS
            msgs = []
            for h in range(HEADS):
                qh = q[:, :, h * hd:(h + 1) * hd]
                kh = k[:, :, h * hd:(h + 1) * hd]
                vh = v[:, :, h * hd:(h + 1) * hd]
                s = jax.lax.dot_general(
                    qh, kh, (((2,), (2,)), ((0,), (0,))),
                    preferred_element_type=jnp.float32) / (hd ** 0.5)
                s = jnp.where(kmask == 0.0, NEG, s)
                s = s - jnp.max(s, axis=-1, keepdims=True)
                es = jnp.exp(s)
                p = es / jnp.sum(es, axis=-1, keepdims=True)
                msgs.append(jax.lax.dot_general(
                    p, vh, (((2,), (1,)), ((0,), (0,))),
                    preferred_element_type=jnp.float32))
            msg = jnp.concatenate(msgs, axis=-1).reshape(BN, FDIM)
            msg = _lin(msg, merge[0], merge[1])
            y = jnp.concatenate([x, msg], axis=-1)
            x = x + _mlp(y, mlp)

        ge = _lin(x, fproj[0], fproj[1])
        cls = _lin(ge, chead[0], chead[1])
        lcls = cls - _logsumexp(cls, axis=-1)
        logcls_ref[...] = lcls.reshape(B, MAXV, NUM_CLASSES - 1)

        geb = ge.reshape(B, MAXV, FDIM)
        mt = jax.lax.dot_general(
            geb, geb, (((2,), (2,)), ((0,), (0,))),
            preferred_element_type=jnp.float32) / (FDIM ** 0.5)
        ii = jax.lax.broadcasted_iota(jnp.int32, (1, MAXV, MAXV), 1)
        jj = jax.lax.broadcasted_iota(jnp.int32, (1, MAXV, MAXV), 2)
        mt = jnp.where(ii == jj, NEG, mt)
        mm = vmask[:, :, None] * vmask[:, None, :]
        mt = jnp.where(mm == 0.0, NEG, mt)

        m = n = MAXV
        acol = jnp.broadcast_to(alpha, (B, m, 1)).astype(jnp.float32)
        arow = jnp.broadcast_to(alpha, (B, 1, n + 1)).astype(jnp.float32)
        couplings = jnp.concatenate(
            [jnp.concatenate([mt, acol], axis=2), arow], axis=1)
        norm = -float(np.log(m + n))
        pi = jax.lax.broadcasted_iota(jnp.int32, (B, m + 1, 1), 1)
        pj = jax.lax.broadcasted_iota(jnp.int32, (B, 1, n + 1), 2)
        lrow = jnp.where(pi < m, norm, float(np.log(n)) + norm)
        lcol = jnp.where(pj < n, norm, float(np.log(m)) + norm)
        u = jnp.zeros((B, m + 1, 1), jnp.float32)
        v = jnp.zeros((B, 1, n + 1), jnp.float32)
        for _ in range(SINK_ITERS):
            u = lrow - _logsumexp(couplings + v, axis=2)
            v = lcol - _logsumexp(couplings + u, axis=1)
        matches_ref[...] = couplings + u + v - norm

    return _stage2_kernel


def _flatten_params(params):
    flat = []
    for ly in params['venc']:
        flat += [ly['W'], ly['b'], ly['gamma'], ly['beta']]
    for ly in params['dtenc']:
        flat += [ly['W'], ly['b'], ly['gamma'], ly['beta']]
    perm = np.array([(kk % (FDIM // HEADS)) * HEADS + kk // (FDIM // HEADS)
                     for kk in range(FDIM)])
    for layer in params['gnn']:
        for nm in ('q', 'k', 'v'):
            flat += [layer[nm]['W'][perm, :], layer[nm]['b'][perm]]
        flat += [layer['merge']['W'][:, perm], layer['merge']['b']]
        for ly in layer['mlp']:
            flat += [ly['W'], ly['b'], ly['gamma'], ly['beta']]
    flat += [params['final_proj']['W'], params['final_proj']['b']]
    flat += [params['cls_head']['W'], params['cls_head']['b']]
    flat += [params['bin_score'].reshape(1)]
    return flat


def _run_stage2(top_s, rows_i, cols_i, vmask, dtc_flat, params):
    wflat = _flatten_params(params)
    n_w = len(wflat)
    out = pl.pallas_call(
        _make_stage2_kernel(n_w),
        out_shape=[
            jax.ShapeDtypeStruct((B, MAXV, NUM_CLASSES - 1), jnp.float32),
            jax.ShapeDtypeStruct((B, MAXV + 1, MAXV + 1), jnp.float32),
        ],
    )(top_s, rows_i, cols_i, vmask, dtc_flat, *wflat)
    return out


def kernel(semantic, distance, vertex, instance, direction, params):
    top_s, rows_i, cols_i, vmask = _run_stage1(vertex)

    dtc_flat = distance.reshape(B, NUM_CLASSES - 1, Hc, CELL, Wc, CELL)
    dtc_flat = dtc_flat.transpose(0, 2, 4, 1, 3, 5).reshape(B, Hc * Wc, DT_DIM)

    lcls, matches = _run_stage2(top_s, rows_i, cols_i, vmask, dtc_flat, params)

    log_cls = lcls.transpose(0, 2, 1)
    vertices = jnp.stack([cols_i, rows_i], axis=-1)
    masks = vmask[:, :, None]
    return (log_cls, distance, vertex, instance, direction, matches,
            vertices, masks)
